# Initial kernel scaffold; baseline (speedup 1.0000x reference)
import jax
import jax.numpy as jnp
from jax.experimental import pallas as pl


def kernel(x, Wr, br, gamma, beta, Wfc, bfc, Wproj, bproj):
    raise NotImplementedError("write your pallas kernel here")


# speedup vs baseline: 2.2520x; 2.2520x over previous
import functools

import jax
import jax.numpy as jnp
from jax import lax
from jax.experimental import pallas as pl
from jax.experimental.pallas import tpu as pltpu
from jax.experimental.pallas import tpu_sc as plsc

HID = 768
INNER = 3072
NE = 64
NT = 2048
NA = 2 * NT
EPS = 1e-05
BLK = 128
NB = 96
P = NB * BLK
KTILE = 768
KT = INNER // KTILE
NC = 2
NS = 16
NW = NC * NS


def _gelu(v):
    return 0.5 * v * (1.0 + jnp.tanh(jnp.sqrt(2.0 / jnp.pi) * (v + 0.044715 * v ** 3)))


def _route_body(x_ref, wr_ref, br_ref, rs_ref, xn_ref, d1_ref, d2_ref,
                w1_ref, w2_ref, be_ref, lv_ref):
    xv = x_ref[...]
    logits = jnp.dot(xv, wr_ref[...], preferred_element_type=jnp.float32) + br_ref[...]
    mx = jnp.max(logits, axis=1, keepdims=True)
    ex = jnp.exp(logits - mx)
    rs = ex / jnp.sum(ex, axis=1, keepdims=True)
    rs_ref[...] = rs

    mu = jnp.mean(xv, axis=1, keepdims=True)
    var = jnp.mean((xv - mu) ** 2, axis=1, keepdims=True)
    xn_ref[...] = (xv - mu) / jnp.sqrt(var + EPS)

    lane = lax.broadcasted_iota(jnp.int32, (NT, NE), 1)
    m1 = jnp.max(rs, axis=1, keepdims=True)
    i1 = jnp.min(jnp.where(rs == m1, lane, NE), axis=1, keepdims=True)
    rs2 = jnp.where(lane == i1, -1.0, rs)
    m2 = jnp.max(rs2, axis=1, keepdims=True)
    i2 = jnp.min(jnp.where(rs2 == m2, lane, NE), axis=1, keepdims=True)
    ssum = m1 + m2
    w1_ref[...] = m1 / ssum
    w2_ref[...] = m2 / ssum

    one1 = (lane == i1).astype(jnp.float32)
    one2 = (lane == i2).astype(jnp.float32)

    def excl_cumsum_rows(m):
        c = m
        s = 1
        while s < NT:
            c = c + jnp.concatenate(
                [jnp.zeros((s, NE), jnp.float32), c[:-s, :]], axis=0)
            s *= 2
        return c - m

    c1 = excl_cumsum_rows(one1)
    tot1 = jnp.sum(one1, axis=0, keepdims=True)
    c2 = excl_cumsum_rows(one2) + tot1
    counts = tot1 + jnp.sum(one2, axis=0, keepdims=True)
    nblk = jnp.floor((counts + (BLK - 1)) * (1.0 / BLK))

    def excl_cumsum_lanes(v):
        c = v
        s = 1
        while s < NE:
            c = c + jnp.concatenate(
                [jnp.zeros((1, s), jnp.float32), c[:, :-s]], axis=1)
            s *= 2
        return c - v

    blkoff = excl_cumsum_lanes(nblk)
    poff = blkoff * float(BLK)
    d1_ref[...] = jnp.sum(one1 * (c1 + poff), axis=1, keepdims=True).astype(jnp.int32)
    d2_ref[...] = jnp.sum(one2 * (c2 + poff), axis=1, keepdims=True).astype(jnp.int32)

    bio = lax.broadcasted_iota(jnp.int32, (NB, NE), 0).astype(jnp.float32)
    eio = lax.broadcasted_iota(jnp.int32, (NB, NE), 1)
    be_ref[...] = jnp.max(jnp.where(blkoff <= bio, eio, 0), axis=1, keepdims=True)
    nlive = jnp.sum(nblk, axis=1, keepdims=True)
    lv_ref[...] = (lax.broadcasted_iota(jnp.int32, (NB, 1), 0).astype(jnp.float32)
                   < nlive).astype(jnp.int32)


def _route(x2, Wr, br):
    f32 = jnp.float32
    i32 = jnp.int32
    return pl.pallas_call(
        _route_body,
        out_shape=[
            jax.ShapeDtypeStruct((NT, NE), f32),
            jax.ShapeDtypeStruct((NT, HID), f32),
            jax.ShapeDtypeStruct((NT, 1), i32),
            jax.ShapeDtypeStruct((NT, 1), i32),
            jax.ShapeDtypeStruct((NT, 1), f32),
            jax.ShapeDtypeStruct((NT, 1), f32),
            jax.ShapeDtypeStruct((NB, 1), i32),
            jax.ShapeDtypeStruct((NB, 1), i32),
        ],
    )(x2, Wr, br.reshape(1, NE))


def _dispatch_build(dflat, tsnf):
    @functools.partial(
        pl.kernel,
        out_type=[jax.ShapeDtypeStruct((P,), jnp.int32),
                  jax.ShapeDtypeStruct((P,), jnp.float32)],
        mesh=plsc.VectorSubcoreMesh(core_axis_name="c", subcore_axis_name="s"),
        compiler_params=pltpu.CompilerParams(needs_layout_passes=False),
        scratch_types=[pltpu.VMEM((NA,), jnp.int32),
                       pltpu.VMEM((NA,), jnp.float32),
                       pltpu.VMEM((P,), jnp.int32),
                       pltpu.VMEM((P,), jnp.float32)],
    )
    def k(d_hbm, t_hbm, gi_hbm, wv_hbm, d_v, t_v, gi_v, wv_v):
        wid = lax.axis_index("s") * NC + lax.axis_index("c")

        @pl.when(wid == 0)
        def _():
            pltpu.sync_copy(d_hbm, d_v)
            pltpu.sync_copy(t_hbm, t_v)
            zi = jnp.zeros((16,), jnp.int32)
            zf = jnp.zeros((16,), jnp.float32)

            def init(i, carry):
                gi_v[pl.ds(i * 16, 16)] = zi
                wv_v[pl.ds(i * 16, 16)] = zf
                return carry

            lax.fori_loop(0, P // 16, init, 0)
            li = lax.iota(jnp.int32, 16)

            def scat(i, carry):
                dd = d_v[pl.ds(i * 16, 16)]
                tok = (i * 16 + li) & (NT - 1)
                plsc.store_scatter(gi_v, [dd], tok)
                plsc.store_scatter(wv_v, [dd], t_v[pl.ds(i * 16, 16)])
                return carry

            lax.fori_loop(0, NA // 16, scat, 0)
            pltpu.sync_copy(gi_v, gi_hbm)
            pltpu.sync_copy(wv_v, wv_hbm)

    return k(dflat, tsnf)


def _gather_rows(gidx, xn):
    rpw = P // NW
    ch = 64

    @functools.partial(
        pl.kernel,
        out_type=jax.ShapeDtypeStruct((P, HID), jnp.float32),
        mesh=plsc.VectorSubcoreMesh(core_axis_name="c", subcore_axis_name="s"),
        compiler_params=pltpu.CompilerParams(needs_layout_passes=False),
        scratch_types=[pltpu.VMEM((ch,), jnp.int32),
                       pltpu.VMEM((ch, HID), jnp.float32),
                       pltpu.SemaphoreType.DMA],
    )
    def k(gi_hbm, xn_hbm, px_hbm, idx_v, rows_v, sem):
        wid = lax.axis_index("s") * NC + lax.axis_index("c")

        def chunk(i, carry):
            base = wid * rpw + i * ch
            pltpu.sync_copy(gi_hbm.at[pl.ds(base, ch)], idx_v)
            pltpu.async_copy(xn_hbm.at[idx_v], rows_v, sem).wait()
            pltpu.sync_copy(rows_v, px_hbm.at[pl.ds(base, ch)])
            return carry

        lax.fori_loop(0, rpw // ch, chunk, 0)

    return k(gidx, xn)


def _ffn_body(be_ref, lv_ref, x_ref, wfc_ref, wpj_ref, g_ref, bta_ref,
              bfc_ref, bpj_ref, wv_ref, out_ref):
    k = pl.program_id(0)
    b = pl.program_id(1)

    @pl.when(lv_ref[b] > 0)
    def _():
        rows = pl.ds(b * BLK, BLK)
        cs = x_ref[...] * g_ref[0] + bta_ref[0]
        a = jnp.dot(cs, wfc_ref[0], preferred_element_type=jnp.float32) + bfc_ref[0]
        a = _gelu(a)
        o = jnp.dot(a, wpj_ref[0], preferred_element_type=jnp.float32)

        @pl.when(k == 0)
        def _():
            out_ref[rows, :] = o + bpj_ref[0]

        @pl.when(k > 0)
        def _():
            out_ref[rows, :] = out_ref[rows, :] + o

        @pl.when(k == KT - 1)
        def _():
            out_ref[rows, :] = out_ref[rows, :] * wv_ref[...]


def _ffn(be, lv, px, Wfc, Wproj, gamma, beta, bfc, bproj, wvec):
    grid_spec = pltpu.PrefetchScalarGridSpec(
        num_scalar_prefetch=2,
        grid=(KT, NB),
        in_specs=[
            pl.BlockSpec((BLK, HID), lambda k, b, be, lv: (b, 0)),
            pl.BlockSpec((1, HID, KTILE), lambda k, b, be, lv: (be[b], 0, k)),
            pl.BlockSpec((1, KTILE, HID), lambda k, b, be, lv: (be[b], k, 0)),
            pl.BlockSpec((1, 1, HID), lambda k, b, be, lv: (be[b], 0, 0)),
            pl.BlockSpec((1, 1, HID), lambda k, b, be, lv: (be[b], 0, 0)),
            pl.BlockSpec((1, 1, KTILE), lambda k, b, be, lv: (be[b], 0, k)),
            pl.BlockSpec((1, 1, HID), lambda k, b, be, lv: (be[b], 0, 0)),
            pl.BlockSpec((BLK, 1), lambda k, b, be, lv: (b, 0)),
        ],
        out_specs=pl.BlockSpec((P, HID), lambda k, b, be, lv: (0, 0)),
    )
    return pl.pallas_call(
        _ffn_body,
        grid_spec=grid_spec,
        out_shape=jax.ShapeDtypeStruct((P, HID), jnp.float32),
    )(be, lv, px, Wfc, Wproj, gamma.reshape(NE, 1, HID), beta.reshape(NE, 1, HID),
      bfc.reshape(NE, 1, INNER), bproj.reshape(NE, 1, HID), wvec)


def _combine(dd0, dd1, pout):
    tpw = NT // NW

    @functools.partial(
        pl.kernel,
        out_type=jax.ShapeDtypeStruct((NT, HID), jnp.float32),
        mesh=plsc.VectorSubcoreMesh(core_axis_name="c", subcore_axis_name="s"),
        compiler_params=pltpu.CompilerParams(needs_layout_passes=False),
        scratch_types=[pltpu.VMEM((tpw,), jnp.int32),
                       pltpu.VMEM((tpw,), jnp.int32),
                       pltpu.VMEM((tpw, HID), jnp.float32),
                       pltpu.VMEM((tpw, HID), jnp.float32),
                       pltpu.SemaphoreType.DMA,
                       pltpu.SemaphoreType.DMA],
    )
    def k(d0_hbm, d1_hbm, po_hbm, out_hbm, i0_v, i1_v, a_v, b_v, s0, s1):
        wid = lax.axis_index("s") * NC + lax.axis_index("c")
        base = wid * tpw
        pltpu.sync_copy(d0_hbm.at[pl.ds(base, tpw)], i0_v)
        pltpu.sync_copy(d1_hbm.at[pl.ds(base, tpw)], i1_v)
        cp0 = pltpu.async_copy(po_hbm.at[i0_v], a_v, s0)
        cp1 = pltpu.async_copy(po_hbm.at[i1_v], b_v, s1)
        cp0.wait()
        cp1.wait()

        def row(r, carry):
            for c in range(HID // 16):
                sl = pl.ds(c * 16, 16)
                a_v[r, sl] = a_v[r, sl] + b_v[r, sl]
            return carry

        lax.fori_loop(0, tpw, row, 0)
        pltpu.sync_copy(a_v, out_hbm.at[pl.ds(base, tpw)])

    return k(dd0, dd1, pout)


def kernel(x, Wr, br, gamma, beta, Wfc, bfc, Wproj, bproj):
    bsz, q_len, d = x.shape
    x2 = x.reshape(NT, HID)
    rs, xn, d1, d2, w1, w2, be, lv = _route(x2, Wr, br)
    dflat = jnp.concatenate([d1[:, 0], d2[:, 0]])
    tsnf = jnp.concatenate([w1[:, 0], w2[:, 0]])
    gidx, wvec = _dispatch_build(dflat, tsnf)
    px = _gather_rows(gidx, xn)
    pout = _ffn(be[:, 0], lv[:, 0], px, Wfc, Wproj, gamma, beta, bfc, bproj,
                wvec.reshape(P, 1))
    out = _combine(d1[:, 0], d2[:, 0], pout)
    return out.reshape(bsz, q_len, d), rs.reshape(bsz, q_len, NE)


# speedup vs baseline: 2.7461x; 1.2194x over previous
import functools

import jax
import jax.numpy as jnp
from jax import lax
from jax.experimental import pallas as pl
from jax.experimental.pallas import tpu as pltpu
from jax.experimental.pallas import tpu_sc as plsc

HID = 768
INNER = 3072
NE = 64
NT = 2048
NA = 2 * NT
EPS = 1e-05
BLK = 128
NB = 96
P = NB * BLK
KTILE = 768
KT = INNER // KTILE
NC = 2
NS = 16
NW = NC * NS


def _gelu(v):
    return 0.5 * v * (1.0 + jnp.tanh(jnp.sqrt(2.0 / jnp.pi) * (v + 0.044715 * v ** 3)))


def _route_body(x_ref, wr_ref, br_ref, rs_ref, xn_ref, d1_ref, d2_ref,
                w1_ref, w2_ref, be_ref, lv_ref, nl_ref):
    xv = x_ref[...]
    logits = jnp.dot(xv, wr_ref[...], preferred_element_type=jnp.float32) + br_ref[...]
    mx = jnp.max(logits, axis=1, keepdims=True)
    ex = jnp.exp(logits - mx)
    rs = ex / jnp.sum(ex, axis=1, keepdims=True)
    rs_ref[...] = rs

    mu = jnp.mean(xv, axis=1, keepdims=True)
    var = jnp.mean((xv - mu) ** 2, axis=1, keepdims=True)
    xn_ref[...] = (xv - mu) / jnp.sqrt(var + EPS)

    lane = lax.broadcasted_iota(jnp.int32, (NT, NE), 1)
    m1 = jnp.max(rs, axis=1, keepdims=True)
    i1 = jnp.min(jnp.where(rs == m1, lane, NE), axis=1, keepdims=True)
    rs2 = jnp.where(lane == i1, -1.0, rs)
    m2 = jnp.max(rs2, axis=1, keepdims=True)
    i2 = jnp.min(jnp.where(rs2 == m2, lane, NE), axis=1, keepdims=True)
    ssum = m1 + m2
    w1_ref[...] = m1 / ssum
    w2_ref[...] = m2 / ssum

    one1 = (lane == i1).astype(jnp.float32)
    one2 = (lane == i2).astype(jnp.float32)

    def excl_cumsum_rows(m):
        c = m
        s = 1
        while s < NT:
            c = c + jnp.concatenate(
                [jnp.zeros((s, NE), jnp.float32), c[:-s, :]], axis=0)
            s *= 2
        return c - m

    c1 = excl_cumsum_rows(one1)
    tot1 = jnp.sum(one1, axis=0, keepdims=True)
    c2 = excl_cumsum_rows(one2) + tot1
    counts = tot1 + jnp.sum(one2, axis=0, keepdims=True)
    nblk = jnp.floor((counts + (BLK - 1)) * (1.0 / BLK))

    def excl_cumsum_lanes(v):
        c = v
        s = 1
        while s < NE:
            c = c + jnp.concatenate(
                [jnp.zeros((1, s), jnp.float32), c[:, :-s]], axis=1)
            s *= 2
        return c - v

    blkoff = excl_cumsum_lanes(nblk)
    poff = blkoff * float(BLK)
    d1_ref[...] = jnp.sum(one1 * (c1 + poff), axis=1, keepdims=True).astype(jnp.int32)
    d2_ref[...] = jnp.sum(one2 * (c2 + poff), axis=1, keepdims=True).astype(jnp.int32)

    bio = lax.broadcasted_iota(jnp.int32, (NB, NE), 0).astype(jnp.float32)
    eio = lax.broadcasted_iota(jnp.int32, (NB, NE), 1)
    be_ref[...] = jnp.max(jnp.where(blkoff <= bio, eio, 0), axis=1, keepdims=True)
    nlive = jnp.sum(nblk, axis=1, keepdims=True)
    lv_ref[...] = (lax.broadcasted_iota(jnp.int32, (NB, 1), 0).astype(jnp.float32)
                   < nlive).astype(jnp.int32)
    nl_ref[...] = jnp.broadcast_to(nlive, (1, 16)).astype(jnp.int32)


def _route(x2, Wr, br):
    f32 = jnp.float32
    i32 = jnp.int32
    return pl.pallas_call(
        _route_body,
        out_shape=[
            jax.ShapeDtypeStruct((NT, NE), f32),
            jax.ShapeDtypeStruct((NT, HID), f32),
            jax.ShapeDtypeStruct((NT, 1), i32),
            jax.ShapeDtypeStruct((NT, 1), i32),
            jax.ShapeDtypeStruct((NT, 1), f32),
            jax.ShapeDtypeStruct((NT, 1), f32),
            jax.ShapeDtypeStruct((NB, 1), i32),
            jax.ShapeDtypeStruct((NB, 1), i32),
            jax.ShapeDtypeStruct((1, 16), i32),
        ],
    )(x2, Wr, br.reshape(1, NE))


def _dispatch_build(dflat, tsnf):
    @functools.partial(
        pl.kernel,
        out_type=[jax.ShapeDtypeStruct((P,), jnp.int32),
                  jax.ShapeDtypeStruct((P,), jnp.float32)],
        mesh=plsc.VectorSubcoreMesh(core_axis_name="c", subcore_axis_name="s"),
        compiler_params=pltpu.CompilerParams(needs_layout_passes=False),
        scratch_types=[pltpu.VMEM((NA,), jnp.int32),
                       pltpu.VMEM((NA,), jnp.float32),
                       pltpu.VMEM((P,), jnp.int32),
                       pltpu.VMEM((P,), jnp.float32)],
    )
    def k(d_hbm, t_hbm, gi_hbm, wv_hbm, d_v, t_v, gi_v, wv_v):
        wid = lax.axis_index("s") * NC + lax.axis_index("c")

        @pl.when(wid == 0)
        def _():
            pltpu.sync_copy(d_hbm, d_v)
            pltpu.sync_copy(t_hbm, t_v)
            zi = jnp.zeros((16,), jnp.int32)
            zf = jnp.zeros((16,), jnp.float32)

            def init(i, carry):
                gi_v[pl.ds(i * 16, 16)] = zi
                wv_v[pl.ds(i * 16, 16)] = zf
                return carry

            lax.fori_loop(0, P // 16, init, 0)
            li = lax.iota(jnp.int32, 16)

            def scat(i, carry):
                dd = d_v[pl.ds(i * 16, 16)]
                tok = (i * 16 + li) & (NT - 1)
                plsc.store_scatter(gi_v, [dd], tok)
                plsc.store_scatter(wv_v, [dd], t_v[pl.ds(i * 16, 16)])
                return carry

            lax.fori_loop(0, NA // 16, scat, 0)
            pltpu.sync_copy(gi_v, gi_hbm)
            pltpu.sync_copy(wv_v, wv_hbm)

    return k(dflat, tsnf)


def _gather_rows(gidx, xn, nlv):
    ch = 48
    cpw = P // (ch * NW)

    @functools.partial(
        pl.kernel,
        out_type=jax.ShapeDtypeStruct((P, HID), jnp.float32),
        mesh=plsc.VectorSubcoreMesh(core_axis_name="c", subcore_axis_name="s"),
        compiler_params=pltpu.CompilerParams(needs_layout_passes=False),
        scratch_types=[pltpu.VMEM((16,), jnp.int32),
                       pltpu.VMEM((ch * cpw,), jnp.int32),
                       pltpu.VMEM((ch, HID), jnp.float32),
                       pltpu.VMEM((ch, HID), jnp.float32),
                       pltpu.VMEM((ch, HID), jnp.float32),
                       pltpu.SemaphoreType.DMA,
                       pltpu.SemaphoreType.DMA,
                       pltpu.SemaphoreType.DMA,
                       pltpu.SemaphoreType.DMA,
                       pltpu.SemaphoreType.DMA,
                       pltpu.SemaphoreType.DMA,
                       pltpu.SemaphoreType.DMA],
    )
    def k(gi_hbm, xn_hbm, nl_hbm, px_hbm, nl_v, idx_v, b0, b1, b2,
          si, g0, g1, g2, w0, w1, w2):
        wid = lax.axis_index("s") * NC + lax.axis_index("c")
        pltpu.sync_copy(nl_hbm, nl_v)
        nl = jnp.max(nl_v[...]) * BLK
        bufs = (b0, b1, b2)
        gs = (g0, g1, g2)
        ws = (w0, w1, w2)
        conds = [(wid + j * NW) * ch < nl for j in range(cpw)]
        icp = [pltpu.make_async_copy(gi_hbm.at[pl.ds((wid + j * NW) * ch, ch)],
                                     idx_v.at[pl.ds(j * ch, ch)], si)
               for j in range(cpw)]
        gd = [pltpu.make_async_copy(xn_hbm.at[idx_v.at[pl.ds(j * ch, ch)]],
                                    bufs[j % 3], gs[j % 3])
              for j in range(cpw)]
        wd = [pltpu.make_async_copy(bufs[j % 3],
                                    px_hbm.at[pl.ds((wid + j * NW) * ch, ch)],
                                    ws[j % 3])
              for j in range(cpw)]
        for j in range(cpw):
            pl.when(conds[j])(lambda j=j: icp[j].start())
        for j in range(cpw):
            pl.when(conds[j])(lambda j=j: icp[j].wait())
        for j in range(min(3, cpw)):
            pl.when(conds[j])(lambda j=j: gd[j].start())
        for j in range(cpw):
            def step(j=j):
                gd[j].wait()
                wd[j].start()
                if j + 3 < cpw:
                    def refill(j=j):
                        wd[j].wait()
                        gd[j + 3].start()
                    pl.when(conds[j + 3])(refill)
                    def drain_only(j=j):
                        wd[j].wait()
                    pl.when(jnp.logical_not(conds[j + 3]))(drain_only)
                else:
                    wd[j].wait()
            pl.when(conds[j])(step)

    return k(gidx, xn, nlv)


def _ffn_body(be_ref, lv_ref, x_ref, wfc_ref, wpj_ref, g_ref, bta_ref,
              bfc_ref, bpj_ref, wv_ref, out_ref):
    k = pl.program_id(0)
    b = pl.program_id(1)

    @pl.when(lv_ref[b] > 0)
    def _():
        rows = pl.ds(b * BLK, BLK)
        cs = x_ref[...] * g_ref[0] + bta_ref[0]
        a = jnp.dot(cs, wfc_ref[0], preferred_element_type=jnp.float32) + bfc_ref[0]
        a = _gelu(a)
        o = jnp.dot(a, wpj_ref[0], preferred_element_type=jnp.float32)

        @pl.when(k == 0)
        def _():
            out_ref[rows, :] = o + bpj_ref[0]

        @pl.when(k > 0)
        def _():
            out_ref[rows, :] = out_ref[rows, :] + o

        @pl.when(k == KT - 1)
        def _():
            out_ref[rows, :] = out_ref[rows, :] * wv_ref[...]


def _ffn(be, lv, px, Wfc, Wproj, gamma, beta, bfc, bproj, wvec):
    grid_spec = pltpu.PrefetchScalarGridSpec(
        num_scalar_prefetch=2,
        grid=(KT, NB),
        in_specs=[
            pl.BlockSpec((BLK, HID), lambda k, b, be, lv: (b, 0)),
            pl.BlockSpec((1, HID, KTILE), lambda k, b, be, lv: (be[b], 0, k)),
            pl.BlockSpec((1, KTILE, HID), lambda k, b, be, lv: (be[b], k, 0)),
            pl.BlockSpec((1, 1, HID), lambda k, b, be, lv: (be[b], 0, 0)),
            pl.BlockSpec((1, 1, HID), lambda k, b, be, lv: (be[b], 0, 0)),
            pl.BlockSpec((1, 1, KTILE), lambda k, b, be, lv: (be[b], 0, k)),
            pl.BlockSpec((1, 1, HID), lambda k, b, be, lv: (be[b], 0, 0)),
            pl.BlockSpec((BLK, 1), lambda k, b, be, lv: (b, 0)),
        ],
        out_specs=pl.BlockSpec((P, HID), lambda k, b, be, lv: (0, 0)),
    )
    return pl.pallas_call(
        _ffn_body,
        grid_spec=grid_spec,
        out_shape=jax.ShapeDtypeStruct((P, HID), jnp.float32),
    )(be, lv, px, Wfc, Wproj, gamma.reshape(NE, 1, HID), beta.reshape(NE, 1, HID),
      bfc.reshape(NE, 1, INNER), bproj.reshape(NE, 1, HID), wvec)


def _combine(dd0, dd1, pout):
    tpw = NT // NW

    @functools.partial(
        pl.kernel,
        out_type=jax.ShapeDtypeStruct((NT, HID), jnp.float32),
        mesh=plsc.VectorSubcoreMesh(core_axis_name="c", subcore_axis_name="s"),
        compiler_params=pltpu.CompilerParams(needs_layout_passes=False),
        scratch_types=[pltpu.VMEM((tpw,), jnp.int32),
                       pltpu.VMEM((tpw,), jnp.int32),
                       pltpu.VMEM((tpw, HID), jnp.float32),
                       pltpu.VMEM((tpw, HID), jnp.float32),
                       pltpu.SemaphoreType.DMA,
                       pltpu.SemaphoreType.DMA],
    )
    def k(d0_hbm, d1_hbm, po_hbm, out_hbm, i0_v, i1_v, a_v, b_v, s0, s1):
        wid = lax.axis_index("s") * NC + lax.axis_index("c")
        base = wid * tpw
        pltpu.sync_copy(d0_hbm.at[pl.ds(base, tpw)], i0_v)
        pltpu.sync_copy(d1_hbm.at[pl.ds(base, tpw)], i1_v)
        cp0 = pltpu.async_copy(po_hbm.at[i0_v], a_v, s0)
        cp1 = pltpu.async_copy(po_hbm.at[i1_v], b_v, s1)
        cp0.wait()
        cp1.wait()

        def row(r, carry):
            for c in range(HID // 16):
                sl = pl.ds(c * 16, 16)
                a_v[r, sl] = a_v[r, sl] + b_v[r, sl]
            return carry

        lax.fori_loop(0, tpw, row, 0)
        pltpu.sync_copy(a_v, out_hbm.at[pl.ds(base, tpw)])

    return k(dd0, dd1, pout)


def kernel(x, Wr, br, gamma, beta, Wfc, bfc, Wproj, bproj):
    bsz, q_len, d = x.shape
    x2 = x.reshape(NT, HID)
    rs, xn, d1, d2, w1, w2, be, lv, nl = _route(x2, Wr, br)
    dflat = jnp.concatenate([d1[:, 0], d2[:, 0]])
    tsnf = jnp.concatenate([w1[:, 0], w2[:, 0]])
    gidx, wvec = _dispatch_build(dflat, tsnf)
    px = _gather_rows(gidx, xn, nl.reshape(16))
    pout = _ffn(be[:, 0], lv[:, 0], px, Wfc, Wproj, gamma, beta, bfc, bproj,
                wvec.reshape(P, 1))
    out = _combine(d1[:, 0], d2[:, 0], pout)
    return out.reshape(bsz, q_len, d), rs.reshape(bsz, q_len, NE)


# speedup vs baseline: 3.2746x; 1.1925x over previous
import functools

import jax
import jax.numpy as jnp
from jax import lax
from jax.experimental import pallas as pl
from jax.experimental.pallas import tpu as pltpu
from jax.experimental.pallas import tpu_sc as plsc

HID = 768
INNER = 3072
NE = 64
NT = 2048
NA = 2 * NT
EPS = 1e-05
BLK = 128
NB = 96
P = NB * BLK
KTILE = 768
KT = INNER // KTILE
NC = 2
NS = 16
NW = NC * NS


def _gelu(v):
    return 0.5 * v * (1.0 + jnp.tanh(jnp.sqrt(2.0 / jnp.pi) * (v + 0.044715 * v ** 3)))


def _route_body(x_ref, wr_ref, br_ref, rs_ref, xn_ref, d1_ref, d2_ref,
                w1_ref, w2_ref, be_ref, lv_ref, nl_ref):
    xv = x_ref[...]
    logits = jnp.dot(xv, wr_ref[...], preferred_element_type=jnp.float32) + br_ref[...]
    mx = jnp.max(logits, axis=1, keepdims=True)
    ex = jnp.exp(logits - mx)
    rs = ex / jnp.sum(ex, axis=1, keepdims=True)
    rs_ref[...] = rs

    mu = jnp.mean(xv, axis=1, keepdims=True)
    var = jnp.mean((xv - mu) ** 2, axis=1, keepdims=True)
    xn_ref[...] = (xv - mu) / jnp.sqrt(var + EPS)

    lane = lax.broadcasted_iota(jnp.int32, (NT, NE), 1)
    m1 = jnp.max(rs, axis=1, keepdims=True)
    i1 = jnp.min(jnp.where(rs == m1, lane, NE), axis=1, keepdims=True)
    rs2 = jnp.where(lane == i1, -1.0, rs)
    m2 = jnp.max(rs2, axis=1, keepdims=True)
    i2 = jnp.min(jnp.where(rs2 == m2, lane, NE), axis=1, keepdims=True)
    ssum = m1 + m2
    w1_ref[...] = m1 / ssum
    w2_ref[...] = m2 / ssum

    one1 = (lane == i1).astype(jnp.float32)
    one2 = (lane == i2).astype(jnp.float32)

    def excl_cumsum_rows(m):
        c = m
        s = 1
        while s < NT:
            c = c + jnp.concatenate(
                [jnp.zeros((s, NE), jnp.float32), c[:-s, :]], axis=0)
            s *= 2
        return c - m

    c1 = excl_cumsum_rows(one1)
    tot1 = jnp.sum(one1, axis=0, keepdims=True)
    c2 = excl_cumsum_rows(one2) + tot1
    counts = tot1 + jnp.sum(one2, axis=0, keepdims=True)
    nblk = jnp.floor((counts + (BLK - 1)) * (1.0 / BLK))

    def excl_cumsum_lanes(v):
        c = v
        s = 1
        while s < NE:
            c = c + jnp.concatenate(
                [jnp.zeros((1, s), jnp.float32), c[:, :-s]], axis=1)
            s *= 2
        return c - v

    blkoff = excl_cumsum_lanes(nblk)
    poff = blkoff * float(BLK)
    d1_ref[...] = jnp.sum(one1 * (c1 + poff), axis=1, keepdims=True).astype(jnp.int32)
    d2_ref[...] = jnp.sum(one2 * (c2 + poff), axis=1, keepdims=True).astype(jnp.int32)

    bio = lax.broadcasted_iota(jnp.int32, (NB, NE), 0).astype(jnp.float32)
    eio = lax.broadcasted_iota(jnp.int32, (NB, NE), 1)
    be_ref[...] = jnp.max(jnp.where(blkoff <= bio, eio, 0), axis=1, keepdims=True)
    nlive = jnp.sum(nblk, axis=1, keepdims=True)
    lv_ref[...] = (lax.broadcasted_iota(jnp.int32, (NB, 1), 0).astype(jnp.float32)
                   < nlive).astype(jnp.int32)
    nl_ref[...] = jnp.broadcast_to(nlive, (1, 16)).astype(jnp.int32)


def _route(x2, Wr, br):
    f32 = jnp.float32
    i32 = jnp.int32
    return pl.pallas_call(
        _route_body,
        out_shape=[
            jax.ShapeDtypeStruct((NT, NE), f32),
            jax.ShapeDtypeStruct((NT, HID), f32),
            jax.ShapeDtypeStruct((NT, 1), i32),
            jax.ShapeDtypeStruct((NT, 1), i32),
            jax.ShapeDtypeStruct((NT, 1), f32),
            jax.ShapeDtypeStruct((NT, 1), f32),
            jax.ShapeDtypeStruct((NB, 1), i32),
            jax.ShapeDtypeStruct((NB, 1), i32),
            jax.ShapeDtypeStruct((1, 16), i32),
        ],
    )(x2, Wr, br.reshape(1, NE))


def _dispatch_build(dflat, tsnf):
    @functools.partial(
        pl.kernel,
        out_type=[jax.ShapeDtypeStruct((P,), jnp.int32),
                  jax.ShapeDtypeStruct((P,), jnp.float32)],
        mesh=plsc.VectorSubcoreMesh(core_axis_name="c", subcore_axis_name="s"),
        compiler_params=pltpu.CompilerParams(needs_layout_passes=False),
        scratch_types=[pltpu.VMEM((NA,), jnp.int32),
                       pltpu.VMEM((NA,), jnp.float32),
                       pltpu.VMEM((P,), jnp.int32),
                       pltpu.VMEM((P,), jnp.float32)],
    )
    def k(d_hbm, t_hbm, gi_hbm, wv_hbm, d_v, t_v, gi_v, wv_v):
        wid = lax.axis_index("s") * NC + lax.axis_index("c")

        @pl.when(wid == 0)
        def _():
            pltpu.sync_copy(d_hbm, d_v)
            pltpu.sync_copy(t_hbm, t_v)
            zi = jnp.zeros((16,), jnp.int32)
            zf = jnp.zeros((16,), jnp.float32)

            def init(i, carry):
                gi_v[pl.ds(i * 16, 16)] = zi
                wv_v[pl.ds(i * 16, 16)] = zf
                return carry

            lax.fori_loop(0, P // 16, init, 0)
            li = lax.iota(jnp.int32, 16)

            def scat(i, carry):
                dd = d_v[pl.ds(i * 16, 16)]
                tok = (i * 16 + li) & (NT - 1)
                plsc.store_scatter(gi_v, [dd], tok)
                plsc.store_scatter(wv_v, [dd], t_v[pl.ds(i * 16, 16)])
                return carry

            lax.fori_loop(0, NA // 16, scat, 0)
            pltpu.sync_copy(gi_v, gi_hbm)
            pltpu.sync_copy(wv_v, wv_hbm)

    return k(dflat, tsnf)


def _gather_rows(gidx, xn, nlv):
    ch = 48
    cpw = P // (ch * NW)

    @functools.partial(
        pl.kernel,
        out_type=jax.ShapeDtypeStruct((P, HID), jnp.float32),
        mesh=plsc.VectorSubcoreMesh(core_axis_name="c", subcore_axis_name="s"),
        compiler_params=pltpu.CompilerParams(needs_layout_passes=False),
        scratch_types=[pltpu.VMEM((16,), jnp.int32),
                       pltpu.VMEM((ch * cpw,), jnp.int32),
                       pltpu.VMEM((ch, HID), jnp.float32),
                       pltpu.VMEM((ch, HID), jnp.float32),
                       pltpu.VMEM((ch, HID), jnp.float32),
                       pltpu.SemaphoreType.DMA,
                       pltpu.SemaphoreType.DMA,
                       pltpu.SemaphoreType.DMA,
                       pltpu.SemaphoreType.DMA,
                       pltpu.SemaphoreType.DMA,
                       pltpu.SemaphoreType.DMA,
                       pltpu.SemaphoreType.DMA],
    )
    def k(gi_hbm, xn_hbm, nl_hbm, px_hbm, nl_v, idx_v, b0, b1, b2,
          si, g0, g1, g2, w0, w1, w2):
        wid = lax.axis_index("s") * NC + lax.axis_index("c")
        pltpu.sync_copy(nl_hbm, nl_v)
        nl = jnp.max(nl_v[...]) * BLK
        bufs = (b0, b1, b2)
        gs = (g0, g1, g2)
        ws = (w0, w1, w2)
        conds = [(wid + j * NW) * ch < nl for j in range(cpw)]
        icp = [pltpu.make_async_copy(gi_hbm.at[pl.ds((wid + j * NW) * ch, ch)],
                                     idx_v.at[pl.ds(j * ch, ch)], si)
               for j in range(cpw)]
        gd = [pltpu.make_async_copy(xn_hbm.at[idx_v.at[pl.ds(j * ch, ch)]],
                                    bufs[j % 3], gs[j % 3])
              for j in range(cpw)]
        wd = [pltpu.make_async_copy(bufs[j % 3],
                                    px_hbm.at[pl.ds((wid + j * NW) * ch, ch)],
                                    ws[j % 3])
              for j in range(cpw)]
        for j in range(cpw):
            pl.when(conds[j])(lambda j=j: icp[j].start())
        for j in range(cpw):
            pl.when(conds[j])(lambda j=j: icp[j].wait())
        for j in range(min(3, cpw)):
            pl.when(conds[j])(lambda j=j: gd[j].start())
        for j in range(cpw):
            def step(j=j):
                gd[j].wait()
                wd[j].start()
                if j + 3 < cpw:
                    def refill(j=j):
                        wd[j].wait()
                        gd[j + 3].start()
                    pl.when(conds[j + 3])(refill)
                    def drain_only(j=j):
                        wd[j].wait()
                    pl.when(jnp.logical_not(conds[j + 3]))(drain_only)
                else:
                    wd[j].wait()
            pl.when(conds[j])(step)

    return k(gidx, xn, nlv)


def _ffn_body(be_ref, lv_ref, x_ref, wfc_ref, wpj_ref, g_ref, bta_ref,
              bfc_ref, bpj_ref, wv_ref, out_ref):
    b = pl.program_id(0)

    @pl.when(lv_ref[b] > 0)
    def _():
        cs = x_ref[...] * g_ref[0] + bta_ref[0]
        a = jnp.dot(cs, wfc_ref[0], preferred_element_type=jnp.float32) + bfc_ref[0]
        a = _gelu(a)
        o = jnp.dot(a, wpj_ref[0], preferred_element_type=jnp.float32)
        out_ref[...] = (o + bpj_ref[0]) * wv_ref[...]


def _ffn(be, lv, px, Wfc, Wproj, gamma, beta, bfc, bproj, wvec):
    grid_spec = pltpu.PrefetchScalarGridSpec(
        num_scalar_prefetch=2,
        grid=(NB,),
        in_specs=[
            pl.BlockSpec((BLK, HID), lambda b, be, lv: (b, 0)),
            pl.BlockSpec((1, HID, INNER), lambda b, be, lv: (be[b], 0, 0)),
            pl.BlockSpec((1, INNER, HID), lambda b, be, lv: (be[b], 0, 0)),
            pl.BlockSpec((1, 1, HID), lambda b, be, lv: (be[b], 0, 0)),
            pl.BlockSpec((1, 1, HID), lambda b, be, lv: (be[b], 0, 0)),
            pl.BlockSpec((1, 1, INNER), lambda b, be, lv: (be[b], 0, 0)),
            pl.BlockSpec((1, 1, HID), lambda b, be, lv: (be[b], 0, 0)),
            pl.BlockSpec((BLK, 1), lambda b, be, lv: (b, 0)),
        ],
        out_specs=pl.BlockSpec((BLK, HID), lambda b, be, lv: (b, 0)),
    )
    return pl.pallas_call(
        _ffn_body,
        grid_spec=grid_spec,
        out_shape=jax.ShapeDtypeStruct((P, HID), jnp.float32),
        compiler_params=pltpu.CompilerParams(
            dimension_semantics=("arbitrary",),
            vmem_limit_bytes=100 * 1024 * 1024,
        ),
    )(be, lv, px, Wfc, Wproj, gamma.reshape(NE, 1, HID), beta.reshape(NE, 1, HID),
      bfc.reshape(NE, 1, INNER), bproj.reshape(NE, 1, HID), wvec)


def _combine(dd0, dd1, pout):
    tpw = NT // NW

    @functools.partial(
        pl.kernel,
        out_type=jax.ShapeDtypeStruct((NT, HID), jnp.float32),
        mesh=plsc.VectorSubcoreMesh(core_axis_name="c", subcore_axis_name="s"),
        compiler_params=pltpu.CompilerParams(needs_layout_passes=False),
        scratch_types=[pltpu.VMEM((tpw,), jnp.int32),
                       pltpu.VMEM((tpw,), jnp.int32),
                       pltpu.VMEM((tpw, HID), jnp.float32),
                       pltpu.VMEM((tpw, HID), jnp.float32),
                       pltpu.SemaphoreType.DMA,
                       pltpu.SemaphoreType.DMA],
    )
    def k(d0_hbm, d1_hbm, po_hbm, out_hbm, i0_v, i1_v, a_v, b_v, s0, s1):
        wid = lax.axis_index("s") * NC + lax.axis_index("c")
        base = wid * tpw
        pltpu.sync_copy(d0_hbm.at[pl.ds(base, tpw)], i0_v)
        pltpu.sync_copy(d1_hbm.at[pl.ds(base, tpw)], i1_v)
        cp0 = pltpu.async_copy(po_hbm.at[i0_v], a_v, s0)
        cp1 = pltpu.async_copy(po_hbm.at[i1_v], b_v, s1)
        cp0.wait()
        cp1.wait()

        def row(r, carry):
            for c in range(HID // 16):
                sl = pl.ds(c * 16, 16)
                a_v[r, sl] = a_v[r, sl] + b_v[r, sl]
            return carry

        lax.fori_loop(0, tpw, row, 0)
        pltpu.sync_copy(a_v, out_hbm.at[pl.ds(base, tpw)])

    return k(dd0, dd1, pout)


def kernel(x, Wr, br, gamma, beta, Wfc, bfc, Wproj, bproj):
    bsz, q_len, d = x.shape
    x2 = x.reshape(NT, HID)
    rs, xn, d1, d2, w1, w2, be, lv, nl = _route(x2, Wr, br)
    dflat = jnp.concatenate([d1[:, 0], d2[:, 0]])
    tsnf = jnp.concatenate([w1[:, 0], w2[:, 0]])
    gidx, wvec = _dispatch_build(dflat, tsnf)
    px = _gather_rows(gidx, xn, nl.reshape(16))
    pout = _ffn(be[:, 0], lv[:, 0], px, Wfc, Wproj, gamma, beta, bfc, bproj,
                wvec.reshape(P, 1))
    out = _combine(d1[:, 0], d2[:, 0], pout)
    return out.reshape(bsz, q_len, d), rs.reshape(bsz, q_len, NE)
